# baseline (speedup 1.0000x reference)
import functools

import jax
import jax.numpy as jnp
import numpy as np
from jax.experimental import pallas as pl

_PACK = 8


def _fwd_kernel(x_ref, c_ref, wbd_ref, r_ref, b_ref, y_ref, stats_ref, *, nb):
    step = pl.program_id(0)
    xp = x_ref[...]
    cp = c_ref[...]
    acc = b_ref[...]
    for b in range(nb):
        m = jnp.dot(xp, wbd_ref[b], preferred_element_type=jnp.float32)
        f = jnp.dot(cp, r_ref[b], preferred_element_type=jnp.float32)
        acc = acc + m * f
    y = jnp.maximum(acc, 0.0)
    y_ref[...] = y

    s1 = jnp.sum(y, axis=0, keepdims=True)
    s2 = jnp.sum(y * y, axis=0, keepdims=True)
    block = jnp.concatenate([s1, s2], axis=0)

    @pl.when(step == 0)
    def _():
        stats_ref[...] = jnp.zeros_like(stats_ref)

    stats_ref[...] += block


def _bn_kernel(y_ref, stats_ref, sum_ref, g_ref, be_ref, o_ref, *, n_rows, cout):
    tot = jnp.dot(stats_ref[...], sum_ref[...], preferred_element_type=jnp.float32)
    mean = tot[0:1, :] * (1.0 / n_rows)
    ex2 = tot[1:2, :] * (1.0 / n_rows)
    var = ex2 - mean * mean
    scale = g_ref[...] * jax.lax.rsqrt(var + 1e-3)
    shift = be_ref[...] - mean * scale
    out = y_ref[...] * scale + shift
    for j in range(_PACK):
        o_ref[j :: _PACK, :] = out[:, cout * j : cout * (j + 1)]


def kernel(input_texture, bary_coeff, num_texture, weights, biases, bn_gamma, bn_beta):
    nt, cin = input_texture.shape
    nb = bary_coeff.shape[1]
    cout = weights.shape[0]
    lanes = _PACK * cout
    ntp = nt // _PACK

    blk = 2048
    nblk = ntp // blk

    xp = input_texture.reshape(ntp, _PACK * cin)
    cp = bary_coeff.reshape(ntp, _PACK * nb)

    eye8 = jnp.eye(_PACK, dtype=jnp.float32)
    w_t = jnp.transpose(weights, (2, 1, 0))
    wbd = jax.vmap(lambda wb: jnp.kron(eye8, wb))(w_t)

    r_np = np.zeros((nb, _PACK * nb, lanes), dtype=np.float32)
    for b in range(nb):
        for j in range(_PACK):
            r_np[b, nb * j + b, cout * j : cout * (j + 1)] = 1.0
    r_all = jnp.asarray(r_np)

    s_sum = jnp.tile(jnp.eye(cout, dtype=jnp.float32), (_PACK, _PACK))

    bias_p = jnp.tile(biases.reshape(1, cout), (1, _PACK))
    gamma_p = jnp.tile(bn_gamma.reshape(1, cout), (1, _PACK))
    beta_p = jnp.tile(bn_beta.reshape(1, cout), (1, _PACK))

    y, stats = pl.pallas_call(
        functools.partial(_fwd_kernel, nb=nb),
        grid=(nblk,),
        in_specs=[
            pl.BlockSpec((blk, lanes), lambda i: (i, 0)),
            pl.BlockSpec((blk, _PACK * nb), lambda i: (i, 0)),
            pl.BlockSpec((nb, lanes, lanes), lambda i: (0, 0, 0)),
            pl.BlockSpec((nb, _PACK * nb, lanes), lambda i: (0, 0, 0)),
            pl.BlockSpec((1, lanes), lambda i: (0, 0)),
        ],
        out_specs=[
            pl.BlockSpec((blk, lanes), lambda i: (i, 0)),
            pl.BlockSpec((2, lanes), lambda i: (0, 0)),
        ],
        out_shape=[
            jax.ShapeDtypeStruct((ntp, lanes), jnp.float32),
            jax.ShapeDtypeStruct((2, lanes), jnp.float32),
        ],
    )(xp, cp, wbd, r_all, bias_p)

    out = pl.pallas_call(
        functools.partial(_bn_kernel, n_rows=float(nt), cout=cout),
        grid=(nblk,),
        in_specs=[
            pl.BlockSpec((blk, lanes), lambda i: (i, 0)),
            pl.BlockSpec((2, lanes), lambda i: (0, 0)),
            pl.BlockSpec((lanes, lanes), lambda i: (0, 0)),
            pl.BlockSpec((1, lanes), lambda i: (0, 0)),
            pl.BlockSpec((1, lanes), lambda i: (0, 0)),
        ],
        out_specs=pl.BlockSpec((blk * _PACK, cout), lambda i: (i, 0)),
        out_shape=jax.ShapeDtypeStruct((nt, cout), jnp.float32),
    )(y, stats, s_sum, gamma_p, beta_p)

    return out


# speedup vs baseline: 10.2706x; 10.2706x over previous
import functools

import jax
import jax.numpy as jnp
from jax.experimental import pallas as pl


def _fwd_kernel(x_ref, c_ref, w_ref, b_ref, y_ref, stats_ref, *, nb, cin):
    step = pl.program_id(0)
    xb = x_ref[...]
    z = jnp.concatenate(
        [xb * c_ref[b : b + 1, :] for b in range(nb)], axis=0
    )
    y = jnp.dot(w_ref[...], z, preferred_element_type=jnp.float32)
    y = jnp.maximum(y + b_ref[:, 0:1], 0.0)
    y_ref[...] = y

    s1 = jnp.sum(y, axis=1, keepdims=True)
    s2 = jnp.sum(y * y, axis=1, keepdims=True)
    block = jnp.concatenate([s1, s2], axis=1)

    @pl.when(step == 0)
    def _():
        stats_ref[...] = jnp.zeros_like(stats_ref)

    stats_ref[:, 0:2] += block


def _bn_kernel(y_ref, stats_ref, g_ref, be_ref, o_ref, *, n_rows):
    s1 = stats_ref[:, 0:1]
    s2 = stats_ref[:, 1:2]
    mean = s1 * (1.0 / n_rows)
    ex2 = s2 * (1.0 / n_rows)
    var = ex2 - mean * mean
    scale = g_ref[:, 0:1] * jax.lax.rsqrt(var + 1e-3)
    shift = be_ref[:, 0:1] - mean * scale
    o_ref[...] = y_ref[...] * scale + shift


def kernel(input_texture, bary_coeff, num_texture, weights, biases, bn_gamma, bn_beta):
    nt, cin = input_texture.shape
    nb = bary_coeff.shape[1]
    cout = weights.shape[0]

    n_cols = 8192
    nblk = nt // n_cols

    xt = input_texture.T
    ct = bary_coeff.T

    w_r = jnp.transpose(weights, (0, 2, 1)).reshape(cout, nb * cin)
    bias_t = jnp.tile(biases.reshape(cout, 1), (1, 128))
    gamma_t = jnp.tile(bn_gamma.reshape(cout, 1), (1, 128))
    beta_t = jnp.tile(bn_beta.reshape(cout, 1), (1, 128))

    y, stats = pl.pallas_call(
        functools.partial(_fwd_kernel, nb=nb, cin=cin),
        grid=(nblk,),
        in_specs=[
            pl.BlockSpec((cin, n_cols), lambda i: (0, i)),
            pl.BlockSpec((nb, n_cols), lambda i: (0, i)),
            pl.BlockSpec((cout, nb * cin), lambda i: (0, 0)),
            pl.BlockSpec((cout, 128), lambda i: (0, 0)),
        ],
        out_specs=[
            pl.BlockSpec((cout, n_cols), lambda i: (0, i)),
            pl.BlockSpec((cout, 128), lambda i: (0, 0)),
        ],
        out_shape=[
            jax.ShapeDtypeStruct((cout, nt), jnp.float32),
            jax.ShapeDtypeStruct((cout, 128), jnp.float32),
        ],
    )(xt, ct, w_r, bias_t)

    out_t = pl.pallas_call(
        functools.partial(_bn_kernel, n_rows=float(nt)),
        grid=(nblk,),
        in_specs=[
            pl.BlockSpec((cout, n_cols), lambda i: (0, i)),
            pl.BlockSpec((cout, 128), lambda i: (0, 0)),
            pl.BlockSpec((cout, 128), lambda i: (0, 0)),
            pl.BlockSpec((cout, 128), lambda i: (0, 0)),
        ],
        out_specs=pl.BlockSpec((cout, n_cols), lambda i: (0, i)),
        out_shape=jax.ShapeDtypeStruct((cout, nt), jnp.float32),
    )(y, stats, gamma_t, beta_t)

    return out_t.T


# speedup vs baseline: 14.9389x; 1.4545x over previous
import functools

import jax
import jax.numpy as jnp
from jax.experimental import pallas as pl
from jax.experimental.pallas import tpu as pltpu


def _fused_kernel(
    x_ref, c_ref, w_ref, b_ref, g_ref, be_ref, o_ref, stats_ref, *, nb, nblk, n_rows
):
    step = pl.program_id(0)

    xb = x_ref[...]
    z = jnp.concatenate([xb * c_ref[b : b + 1, :] for b in range(nb)], axis=0)
    y = jnp.dot(w_ref[...], z, preferred_element_type=jnp.float32)
    y = jnp.maximum(y + b_ref[:, 0:1], 0.0)

    @pl.when(step == 0)
    def _():
        stats_ref[...] = jnp.zeros_like(stats_ref)

    @pl.when(step < nblk)
    def _():
        s1 = jnp.sum(y, axis=1, keepdims=True)
        s2 = jnp.sum(y * y, axis=1, keepdims=True)
        stats_ref[:, 0:2] += jnp.concatenate([s1, s2], axis=1)

    @pl.when(step == nblk - 1)
    def _():
        s1 = stats_ref[:, 0:1]
        s2 = stats_ref[:, 1:2]
        mean = s1 * (1.0 / n_rows)
        var = s2 * (1.0 / n_rows) - mean * mean
        scale = g_ref[:, 0:1] * jax.lax.rsqrt(var + 1e-3)
        shift = be_ref[:, 0:1] - mean * scale
        stats_ref[:, 2:3] = scale
        stats_ref[:, 3:4] = shift

    @pl.when(step >= nblk)
    def _():
        o_ref[...] = y * stats_ref[:, 2:3] + stats_ref[:, 3:4]


def kernel(input_texture, bary_coeff, num_texture, weights, biases, bn_gamma, bn_beta):
    nt, cin = input_texture.shape
    nb = bary_coeff.shape[1]
    cout = weights.shape[0]

    n_cols = 16384
    nblk = nt // n_cols

    xt = input_texture.T
    ct = bary_coeff.T

    w_r = jnp.transpose(weights, (0, 2, 1)).reshape(cout, nb * cin)
    bias_t = jnp.tile(biases.reshape(cout, 1), (1, 128))
    gamma_t = jnp.tile(bn_gamma.reshape(cout, 1), (1, 128))
    beta_t = jnp.tile(bn_beta.reshape(cout, 1), (1, 128))

    def col(i):
        return jnp.where(i < nblk, i, i - nblk)

    out_t = pl.pallas_call(
        functools.partial(_fused_kernel, nb=nb, nblk=nblk, n_rows=float(nt)),
        grid=(2 * nblk,),
        in_specs=[
            pl.BlockSpec((cin, n_cols), lambda i: (0, col(i))),
            pl.BlockSpec((nb, n_cols), lambda i: (0, col(i))),
            pl.BlockSpec((cout, nb * cin), lambda i: (0, 0)),
            pl.BlockSpec((cout, 128), lambda i: (0, 0)),
            pl.BlockSpec((cout, 128), lambda i: (0, 0)),
            pl.BlockSpec((cout, 128), lambda i: (0, 0)),
        ],
        out_specs=pl.BlockSpec(
            (cout, n_cols), lambda i: (0, jnp.where(i < nblk, 0, i - nblk))
        ),
        out_shape=jax.ShapeDtypeStruct((cout, nt), jnp.float32),
        scratch_shapes=[pltpu.VMEM((cout, 128), jnp.float32)],
    )(xt, ct, w_r, bias_t, gamma_t, beta_t)

    return out_t.T


# speedup vs baseline: 21.2814x; 1.4246x over previous
import functools

import jax
import jax.numpy as jnp
from jax.experimental import pallas as pl
from jax.experimental.pallas import tpu as pltpu


def _fused_kernel(
    x_ref,
    c_ref,
    w_ref,
    b_ref,
    g_ref,
    be_ref,
    o_ref,
    ycache_ref,
    stats_ref,
    *,
    nb,
    nblk,
    n_cols,
    n_rows,
):
    step = pl.program_id(0)

    @pl.when(step == 0)
    def _():
        stats_ref[...] = jnp.zeros_like(stats_ref)

    @pl.when(step < nblk)
    def _():
        xb = x_ref[...]
        z = jnp.concatenate([xb * c_ref[b : b + 1, :] for b in range(nb)], axis=0)
        y = jnp.dot(w_ref[...], z, preferred_element_type=jnp.float32)
        y = jnp.maximum(y + b_ref[:, 0:1], 0.0)
        ycache_ref[:, pl.ds(step * n_cols, n_cols)] = y.astype(jnp.bfloat16)
        s1 = jnp.sum(y, axis=1, keepdims=True)
        s2 = jnp.sum(y * y, axis=1, keepdims=True)
        stats_ref[:, 0:2] += jnp.concatenate([s1, s2], axis=1)

    @pl.when(step == nblk - 1)
    def _():
        s1 = stats_ref[:, 0:1]
        s2 = stats_ref[:, 1:2]
        mean = s1 * (1.0 / n_rows)
        var = s2 * (1.0 / n_rows) - mean * mean
        scale = g_ref[:, 0:1] * jax.lax.rsqrt(var + 1e-3)
        shift = be_ref[:, 0:1] - mean * scale
        stats_ref[:, 2:3] = scale
        stats_ref[:, 3:4] = shift

    @pl.when(step >= nblk)
    def _():
        y = ycache_ref[:, pl.ds((step - nblk) * n_cols, n_cols)].astype(jnp.float32)
        o_ref[...] = y * stats_ref[:, 2:3] + stats_ref[:, 3:4]


def kernel(input_texture, bary_coeff, num_texture, weights, biases, bn_gamma, bn_beta):
    nt, cin = input_texture.shape
    nb = bary_coeff.shape[1]
    cout = weights.shape[0]

    n_cols = 16384
    nblk = nt // n_cols

    xt = input_texture.T
    ct = bary_coeff.T

    w_r = jnp.transpose(weights, (0, 2, 1)).reshape(cout, nb * cin)
    bias_t = jnp.tile(biases.reshape(cout, 1), (1, 128))
    gamma_t = jnp.tile(bn_gamma.reshape(cout, 1), (1, 128))
    beta_t = jnp.tile(bn_beta.reshape(cout, 1), (1, 128))

    out_t = pl.pallas_call(
        functools.partial(
            _fused_kernel, nb=nb, nblk=nblk, n_cols=n_cols, n_rows=float(nt)
        ),
        grid=(2 * nblk,),
        in_specs=[
            pl.BlockSpec((cin, n_cols), lambda i: (0, jnp.where(i < nblk, i, 0))),
            pl.BlockSpec((nb, n_cols), lambda i: (0, jnp.where(i < nblk, i, 0))),
            pl.BlockSpec((cout, nb * cin), lambda i: (0, 0)),
            pl.BlockSpec((cout, 128), lambda i: (0, 0)),
            pl.BlockSpec((cout, 128), lambda i: (0, 0)),
            pl.BlockSpec((cout, 128), lambda i: (0, 0)),
        ],
        out_specs=pl.BlockSpec(
            (cout, n_cols), lambda i: (0, jnp.where(i < nblk, 0, i - nblk))
        ),
        out_shape=jax.ShapeDtypeStruct((cout, nt), jnp.float32),
        scratch_shapes=[
            pltpu.VMEM((cout, nt), jnp.bfloat16),
            pltpu.VMEM((cout, 128), jnp.float32),
        ],
    )(xt, ct, w_r, bias_t, gamma_t, beta_t)

    return out_t.T


# speedup vs baseline: 28.7550x; 1.3512x over previous
import functools

import jax
import jax.numpy as jnp
from jax.experimental import pallas as pl
from jax.experimental.pallas import tpu as pltpu


def _fused_kernel(
    x_ref,
    c_ref,
    w_ref,
    b_ref,
    g_ref,
    be_ref,
    o_ref,
    ycache_ref,
    stats_ref,
    *,
    nb,
    nblk,
    n_cols,
    n_rows,
):
    step = pl.program_id(0)

    @pl.when(step == 0)
    def _():
        stats_ref[...] = jnp.zeros_like(stats_ref)

    @pl.when(step < nblk)
    def _():
        xb = x_ref[...]
        z = jnp.concatenate([xb * c_ref[b : b + 1, :] for b in range(nb)], axis=0)
        y = jnp.dot(w_ref[...], z, preferred_element_type=jnp.float32)
        y = jnp.maximum(y + b_ref[:, 0:1], 0.0)
        ycache_ref[:, pl.ds(step * n_cols, n_cols)] = y.astype(jnp.bfloat16)
        s1 = jnp.sum(y, axis=1, keepdims=True)
        s2 = jnp.sum(y * y, axis=1, keepdims=True)
        stats_ref[:, 0:2] += jnp.concatenate([s1, s2], axis=1)

    @pl.when(step == nblk - 1)
    def _():
        s1 = stats_ref[:, 0:1]
        s2 = stats_ref[:, 1:2]
        mean = s1 * (1.0 / n_rows)
        var = s2 * (1.0 / n_rows) - mean * mean
        scale = g_ref[:, 0:1] * jax.lax.rsqrt(var + 1e-3)
        shift = be_ref[:, 0:1] - mean * scale
        stats_ref[:, 2:3] = scale
        stats_ref[:, 3:4] = shift

    @pl.when(step >= nblk)
    def _():
        y = ycache_ref[:, pl.ds((step - nblk) * n_cols, n_cols)].astype(jnp.float32)
        o_ref[...] = y * stats_ref[:, 2:3] + stats_ref[:, 3:4]


def kernel(input_texture, bary_coeff, num_texture, weights, biases, bn_gamma, bn_beta):
    nt, cin = input_texture.shape
    nb = bary_coeff.shape[1]
    cout = weights.shape[0]

    n_cols = 32768
    nblk = nt // n_cols

    xt = input_texture.T
    ct = bary_coeff.T

    w_r = jnp.transpose(weights, (0, 2, 1)).reshape(cout, nb * cin)
    bias_t = jnp.tile(biases.reshape(cout, 1), (1, 128))
    gamma_t = jnp.tile(bn_gamma.reshape(cout, 1), (1, 128))
    beta_t = jnp.tile(bn_beta.reshape(cout, 1), (1, 128))

    out_t = pl.pallas_call(
        functools.partial(
            _fused_kernel, nb=nb, nblk=nblk, n_cols=n_cols, n_rows=float(nt)
        ),
        grid=(2 * nblk,),
        in_specs=[
            pl.BlockSpec((cin, n_cols), lambda i: (0, jnp.where(i < nblk, i, 0))),
            pl.BlockSpec((nb, n_cols), lambda i: (0, jnp.where(i < nblk, i, 0))),
            pl.BlockSpec((cout, nb * cin), lambda i: (0, 0)),
            pl.BlockSpec((cout, 128), lambda i: (0, 0)),
            pl.BlockSpec((cout, 128), lambda i: (0, 0)),
            pl.BlockSpec((cout, 128), lambda i: (0, 0)),
        ],
        out_specs=pl.BlockSpec(
            (cout, n_cols), lambda i: (0, jnp.where(i < nblk, 0, i - nblk))
        ),
        out_shape=jax.ShapeDtypeStruct((cout, nt), jnp.float32),
        scratch_shapes=[
            pltpu.VMEM((cout, nt), jnp.bfloat16),
            pltpu.VMEM((cout, 128), jnp.float32),
        ],
    )(xt, ct, w_r, bias_t, gamma_t, beta_t)

    return out_t.T


# speedup vs baseline: 34.4011x; 1.1964x over previous
import functools

import jax
import jax.numpy as jnp
from jax.experimental import pallas as pl
from jax.experimental.pallas import tpu as pltpu


def _fused_kernel(
    x_ref,
    c_ref,
    w_ref,
    b_ref,
    g_ref,
    be_ref,
    o_ref,
    ycache_ref,
    stats_ref,
    *,
    nb,
    nblk,
    n_cols,
    n_rows,
):
    step = pl.program_id(0)

    @pl.when(step == 0)
    def _():
        stats_ref[...] = jnp.zeros_like(stats_ref)

    @pl.when(step < nblk)
    def _():
        xb = x_ref[...]
        z = jnp.concatenate([xb * c_ref[b : b + 1, :] for b in range(nb)], axis=0)
        y = jnp.dot(w_ref[...], z, preferred_element_type=jnp.float32)
        y = jnp.maximum(y + b_ref[:, 0:1], 0.0)
        ycache_ref[:, pl.ds(step * n_cols, n_cols)] = y.astype(jnp.bfloat16)
        s1 = jnp.sum(y, axis=1, keepdims=True)
        s2 = jnp.sum(y * y, axis=1, keepdims=True)
        stats_ref[:, 0:2] += jnp.concatenate([s1, s2], axis=1)

    @pl.when(step == nblk - 1)
    def _():
        s1 = stats_ref[:, 0:1]
        s2 = stats_ref[:, 1:2]
        mean = s1 * (1.0 / n_rows)
        var = s2 * (1.0 / n_rows) - mean * mean
        scale = g_ref[:, 0:1] * jax.lax.rsqrt(var + 1e-3)
        shift = be_ref[:, 0:1] - mean * scale
        stats_ref[:, 2:3] = scale
        stats_ref[:, 3:4] = shift

    @pl.when(step >= nblk)
    def _():
        y = ycache_ref[:, pl.ds((step - nblk) * n_cols, n_cols)].astype(jnp.float32)
        o_ref[...] = y * stats_ref[:, 2:3] + stats_ref[:, 3:4]


def kernel(input_texture, bary_coeff, num_texture, weights, biases, bn_gamma, bn_beta):
    nt, cin = input_texture.shape
    nb = bary_coeff.shape[1]
    cout = weights.shape[0]

    n_cols = 65536
    nblk = nt // n_cols

    xt = input_texture.T
    ct = bary_coeff.T

    w_r = jnp.transpose(weights, (0, 2, 1)).reshape(cout, nb * cin)
    bias_t = jnp.tile(biases.reshape(cout, 1), (1, 128))
    gamma_t = jnp.tile(bn_gamma.reshape(cout, 1), (1, 128))
    beta_t = jnp.tile(bn_beta.reshape(cout, 1), (1, 128))

    out_t = pl.pallas_call(
        functools.partial(
            _fused_kernel, nb=nb, nblk=nblk, n_cols=n_cols, n_rows=float(nt)
        ),
        grid=(2 * nblk,),
        in_specs=[
            pl.BlockSpec((cin, n_cols), lambda i: (0, jnp.where(i < nblk, i, 0))),
            pl.BlockSpec((nb, n_cols), lambda i: (0, jnp.where(i < nblk, i, 0))),
            pl.BlockSpec((cout, nb * cin), lambda i: (0, 0)),
            pl.BlockSpec((cout, 128), lambda i: (0, 0)),
            pl.BlockSpec((cout, 128), lambda i: (0, 0)),
            pl.BlockSpec((cout, 128), lambda i: (0, 0)),
        ],
        out_specs=pl.BlockSpec(
            (cout, n_cols), lambda i: (0, jnp.where(i < nblk, 0, i - nblk))
        ),
        out_shape=jax.ShapeDtypeStruct((cout, nt), jnp.float32),
        scratch_shapes=[
            pltpu.VMEM((cout, nt), jnp.bfloat16),
            pltpu.VMEM((cout, 128), jnp.float32),
        ],
    )(xt, ct, w_r, bias_t, gamma_t, beta_t)

    return out_t.T
